# initial kernel scaffold (unmeasured)
import jax
import jax.numpy as jnp
from jax import lax
from jax.experimental import pallas as pl
from jax.experimental.pallas import tpu as pltpu

N_DEV = 8
B = 2
SEQ = 4096
TAPS = 4
COUT = 1024
N_CHUNK = 8
ROWS = B * SEQ
CH_ROWS = ROWS // N_CHUNK
SEQ_CH = SEQ // (N_CHUNK // B)
STEPS = N_DEV - 1


def kernel(x, k, Wp):
    cin = x.shape[2]

    def body(x_ref, k_ref, w_ref, out_ref,
             recv_ref, send_sem, rs_sems, ag_sems, credit_sem):
        my = lax.axis_index("i")
        left = lax.rem(my + N_DEV - 1, N_DEV)
        right = lax.rem(my + 1, N_DEV)

        barrier = pltpu.get_barrier_semaphore()
        for nbr in (left, right):
            pl.semaphore_signal(barrier, inc=1, device_id=(nbr,),
                                device_id_type=pl.DeviceIdType.MESH)
        pl.semaphore_wait(barrier, 2)

        kv = k_ref[:, :]
        w = w_ref[:, :]
        for c in range(N_CHUNK):
            b, j = divmod(c, N_CHUNK // B)
            s0 = j * SEQ_CH
            if j == 0:
                xc = x_ref[b, pl.ds(0, SEQ_CH), :]
                xp = jnp.concatenate(
                    [jnp.zeros((TAPS - 1, cin), jnp.float32), xc], axis=0)
            else:
                xp = x_ref[b, pl.ds(s0 - (TAPS - 1), SEQ_CH + TAPS - 1), :]
            y = (xp[3:SEQ_CH + 3] * kv[3]
                 + xp[2:SEQ_CH + 2] * kv[2]
                 + xp[1:SEQ_CH + 1] * kv[1]
                 + xp[0:SEQ_CH] * kv[0])
            a = y * jax.nn.sigmoid(y)
            out_ref[c] = jnp.dot(a, w, preferred_element_type=jnp.float32)

        for s in range(STEPS):
            slot = s % 2
            sc = lax.rem(my - s + N_DEV, N_DEV)
            rc = lax.rem(my - s - 1 + N_DEV, N_DEV)
            if s >= 2:
                pl.semaphore_wait(credit_sem, 1)
            rdma = pltpu.make_async_remote_copy(
                src_ref=out_ref.at[sc],
                dst_ref=recv_ref.at[slot],
                send_sem=send_sem,
                recv_sem=rs_sems.at[slot],
                device_id=(right,),
                device_id_type=pl.DeviceIdType.MESH,
            )
            rdma.start()
            rdma.wait()
            out_ref[rc] = out_ref[rc] + recv_ref[slot]
            if s < STEPS - 2:
                pl.semaphore_signal(credit_sem, inc=1, device_id=(left,),
                                    device_id_type=pl.DeviceIdType.MESH)

        for s in range(STEPS):
            sc = lax.rem(my + 1 - s + N_DEV, N_DEV)
            rdma = pltpu.make_async_remote_copy(
                src_ref=out_ref.at[sc],
                dst_ref=out_ref.at[sc],
                send_sem=send_sem,
                recv_sem=ag_sems.at[s],
                device_id=(right,),
                device_id_type=pl.DeviceIdType.MESH,
            )
            rdma.start()
            rdma.wait()

    out = pl.pallas_call(
        body,
        out_shape=jax.ShapeDtypeStruct((N_CHUNK, CH_ROWS, COUT), jnp.float32),
        in_specs=[
            pl.BlockSpec(memory_space=pltpu.VMEM),
            pl.BlockSpec(memory_space=pltpu.VMEM),
            pl.BlockSpec(memory_space=pltpu.VMEM),
        ],
        out_specs=pl.BlockSpec(memory_space=pltpu.VMEM),
        scratch_shapes=[
            pltpu.VMEM((2, CH_ROWS, COUT), jnp.float32),
            pltpu.SemaphoreType.DMA,
            pltpu.SemaphoreType.DMA((2,)),
            pltpu.SemaphoreType.DMA((STEPS,)),
            pltpu.SemaphoreType.REGULAR,
        ],
        compiler_params=pltpu.CompilerParams(collective_id=0),
    )(x, k, Wp)
    return out.reshape(B, SEQ, COUT)


# baseline (device time: 733448 ns/iter reference)
import jax
import jax.numpy as jnp
from jax import lax
from jax.experimental import pallas as pl
from jax.experimental.pallas import tpu as pltpu

N_DEV = 8
B = 2
SEQ = 4096
TAPS = 4
COUT = 1024
N_CHUNK = 8
ROWS = B * SEQ
CH_ROWS = ROWS // N_CHUNK
STEPS = N_DEV - 1

SUB = 512
NSUB = ROWS // SUB
HALO = 8


def kernel(x, k, Wp):
    cin = x.shape[2]

    def body(x_ref, k_ref, w_ref, out_ref,
             stage_ref, copy_sems, recv_ref, send_sem, rs_sems, ag_sems,
             credit_sem):
        my = lax.axis_index("i")
        left = lax.rem(my + N_DEV - 1, N_DEV)
        right = lax.rem(my + 1, N_DEV)

        barrier = pltpu.get_barrier_semaphore()
        for nbr in (left, right):
            pl.semaphore_signal(barrier, inc=1, device_id=(nbr,),
                                device_id_type=pl.DeviceIdType.MESH)
        pl.semaphore_wait(barrier, 2)

        def issue_copy(n):
            slot = n % 2
            b, j = divmod(n, NSUB // B)
            s0 = j * SUB
            if j == 0:
                stage_ref[slot, 0:HALO, :] = jnp.zeros((HALO, cin), jnp.float32)
                cp = pltpu.make_async_copy(
                    x_ref.at[b, pl.ds(0, SUB), :],
                    stage_ref.at[slot, pl.ds(HALO, SUB), :],
                    copy_sems.at[slot],
                )
            else:
                cp = pltpu.make_async_copy(
                    x_ref.at[b, pl.ds(s0 - HALO, SUB + HALO), :],
                    stage_ref.at[slot, :, :],
                    copy_sems.at[slot],
                )
            cp.start()
            return cp

        kv = k_ref[:, :]
        w = w_ref[:, :]
        cp = issue_copy(0)
        for n in range(NSUB):
            cp.wait()
            if n + 1 < NSUB:
                cp = issue_copy(n + 1)
            slot = n % 2
            xp = stage_ref[slot]
            y = (xp[HALO:HALO + SUB] * kv[3]
                 + xp[HALO - 1:HALO - 1 + SUB] * kv[2]
                 + xp[HALO - 2:HALO - 2 + SUB] * kv[1]
                 + xp[HALO - 3:HALO - 3 + SUB] * kv[0])
            a = y * jax.nn.sigmoid(y)
            c, h = divmod(n, CH_ROWS // SUB)
            out_ref[c, pl.ds(h * SUB, SUB), :] = jnp.dot(
                a, w, preferred_element_type=jnp.float32)

        for s in range(STEPS):
            slot = s % 2
            sc = lax.rem(my - s + N_DEV, N_DEV)
            rc = lax.rem(my - s - 1 + N_DEV, N_DEV)
            if s >= 2:
                pl.semaphore_wait(credit_sem, 1)
            rdma = pltpu.make_async_remote_copy(
                src_ref=out_ref.at[sc],
                dst_ref=recv_ref.at[slot],
                send_sem=send_sem,
                recv_sem=rs_sems.at[slot],
                device_id=(right,),
                device_id_type=pl.DeviceIdType.MESH,
            )
            rdma.start()
            rdma.wait()
            out_ref[rc] = out_ref[rc] + recv_ref[slot]
            if s < STEPS - 2:
                pl.semaphore_signal(credit_sem, inc=1, device_id=(left,),
                                    device_id_type=pl.DeviceIdType.MESH)

        for s in range(STEPS):
            sc = lax.rem(my + 1 - s + N_DEV, N_DEV)
            rdma = pltpu.make_async_remote_copy(
                src_ref=out_ref.at[sc],
                dst_ref=out_ref.at[sc],
                send_sem=send_sem,
                recv_sem=ag_sems.at[s],
                device_id=(right,),
                device_id_type=pl.DeviceIdType.MESH,
            )
            rdma.start()
            rdma.wait()

    out = pl.pallas_call(
        body,
        out_shape=jax.ShapeDtypeStruct((N_CHUNK, CH_ROWS, COUT), jnp.float32),
        in_specs=[
            pl.BlockSpec(memory_space=pltpu.MemorySpace.HBM),
            pl.BlockSpec(memory_space=pltpu.VMEM),
            pl.BlockSpec(memory_space=pltpu.VMEM),
        ],
        out_specs=pl.BlockSpec(memory_space=pltpu.VMEM),
        scratch_shapes=[
            pltpu.VMEM((2, SUB + HALO, cin), jnp.float32),
            pltpu.SemaphoreType.DMA((2,)),
            pltpu.VMEM((2, CH_ROWS, COUT), jnp.float32),
            pltpu.SemaphoreType.DMA,
            pltpu.SemaphoreType.DMA((2,)),
            pltpu.SemaphoreType.DMA((STEPS,)),
            pltpu.SemaphoreType.REGULAR,
        ],
        compiler_params=pltpu.CompilerParams(
            collective_id=0, vmem_limit_bytes=63 * 1024 * 1024),
    )(x, k, Wp)
    return out.reshape(B, SEQ, COUT)


# device time: 424014 ns/iter; 1.7298x vs baseline; 1.7298x over previous
import jax
import jax.numpy as jnp
from jax import lax
from jax.experimental import pallas as pl
from jax.experimental.pallas import tpu as pltpu

N_DEV = 8
B = 2
SEQ = 4096
TAPS = 4
COUT = 1024
N_CHUNK = 8
ROWS = B * SEQ
CH_ROWS = ROWS // N_CHUNK
HALF = CH_ROWS // 2
STEPS = N_DEV - 1

SUB = 512
NSUB = ROWS // SUB
HALO = 8


def kernel(x, k, Wp):
    cin = x.shape[2]

    def body(x_ref, k_ref, w_ref, out_ref,
             stage_ref, copy_sems, rcw_ref, rccw_ref,
             send_sems, rs_sems, ag_sems, credit_sems):
        my = lax.axis_index("i")
        left = lax.rem(my + N_DEV - 1, N_DEV)
        right = lax.rem(my + 1, N_DEV)

        barrier = pltpu.get_barrier_semaphore()
        for nbr in (left, right):
            pl.semaphore_signal(barrier, inc=1, device_id=(nbr,),
                                device_id_type=pl.DeviceIdType.MESH)
        pl.semaphore_wait(barrier, 2)

        def issue_copy(n):
            slot = n % 2
            b, j = divmod(n, NSUB // B)
            s0 = j * SUB
            if j == 0:
                stage_ref[slot, 0:HALO, :] = jnp.zeros((HALO, cin), jnp.float32)
                cp = pltpu.make_async_copy(
                    x_ref.at[b, pl.ds(0, SUB), :],
                    stage_ref.at[slot, pl.ds(HALO, SUB), :],
                    copy_sems.at[slot],
                )
            else:
                cp = pltpu.make_async_copy(
                    x_ref.at[b, pl.ds(s0 - HALO, SUB + HALO), :],
                    stage_ref.at[slot, :, :],
                    copy_sems.at[slot],
                )
            cp.start()
            return cp

        kv = k_ref[:, :]
        w = w_ref[:, :]
        cp = issue_copy(0)
        for n in range(NSUB):
            cp.wait()
            if n + 1 < NSUB:
                cp = issue_copy(n + 1)
            slot = n % 2
            xp = stage_ref[slot]
            y = (xp[HALO:HALO + SUB] * kv[3]
                 + xp[HALO - 1:HALO - 1 + SUB] * kv[2]
                 + xp[HALO - 2:HALO - 2 + SUB] * kv[1]
                 + xp[HALO - 3:HALO - 3 + SUB] * kv[0])
            a = y * jax.nn.sigmoid(y)
            c, h = divmod(n, CH_ROWS // SUB)
            out_ref[c, pl.ds(h * SUB, SUB), :] = jnp.dot(
                a, w, preferred_element_type=jnp.float32)

        for s in range(STEPS):
            slot = s % 2
            sc0 = lax.rem(my - s + N_DEV, N_DEV)
            rc0 = lax.rem(my - s - 1 + N_DEV, N_DEV)
            sc1 = lax.rem(my + s, N_DEV)
            rc1 = lax.rem(my + s + 1, N_DEV)
            if s >= 2:
                pl.semaphore_wait(credit_sems.at[0], 1)
                pl.semaphore_wait(credit_sems.at[1], 1)
            cw = pltpu.make_async_remote_copy(
                src_ref=out_ref.at[sc0, pl.ds(0, HALF), :],
                dst_ref=rcw_ref.at[slot],
                send_sem=send_sems.at[0],
                recv_sem=rs_sems.at[0, slot],
                device_id=(right,),
                device_id_type=pl.DeviceIdType.MESH,
            )
            ccw = pltpu.make_async_remote_copy(
                src_ref=out_ref.at[sc1, pl.ds(HALF, HALF), :],
                dst_ref=rccw_ref.at[slot],
                send_sem=send_sems.at[1],
                recv_sem=rs_sems.at[1, slot],
                device_id=(left,),
                device_id_type=pl.DeviceIdType.MESH,
            )
            cw.start()
            ccw.start()
            cw.wait()
            ccw.wait()
            out_ref[rc0, pl.ds(0, HALF), :] = (
                out_ref[rc0, pl.ds(0, HALF), :] + rcw_ref[slot])
            out_ref[rc1, pl.ds(HALF, HALF), :] = (
                out_ref[rc1, pl.ds(HALF, HALF), :] + rccw_ref[slot])
            if s < STEPS - 2:
                pl.semaphore_signal(credit_sems.at[0], inc=1, device_id=(left,),
                                    device_id_type=pl.DeviceIdType.MESH)
                pl.semaphore_signal(credit_sems.at[1], inc=1, device_id=(right,),
                                    device_id_type=pl.DeviceIdType.MESH)

        for s in range(STEPS):
            sc0 = lax.rem(my + 1 - s + N_DEV, N_DEV)
            sc1 = lax.rem(my - 1 + s + N_DEV, N_DEV)
            cw = pltpu.make_async_remote_copy(
                src_ref=out_ref.at[sc0, pl.ds(0, HALF), :],
                dst_ref=out_ref.at[sc0, pl.ds(0, HALF), :],
                send_sem=send_sems.at[0],
                recv_sem=ag_sems.at[0, s],
                device_id=(right,),
                device_id_type=pl.DeviceIdType.MESH,
            )
            ccw = pltpu.make_async_remote_copy(
                src_ref=out_ref.at[sc1, pl.ds(HALF, HALF), :],
                dst_ref=out_ref.at[sc1, pl.ds(HALF, HALF), :],
                send_sem=send_sems.at[1],
                recv_sem=ag_sems.at[1, s],
                device_id=(left,),
                device_id_type=pl.DeviceIdType.MESH,
            )
            cw.start()
            ccw.start()
            cw.wait()
            ccw.wait()

    out = pl.pallas_call(
        body,
        out_shape=jax.ShapeDtypeStruct((N_CHUNK, CH_ROWS, COUT), jnp.float32),
        in_specs=[
            pl.BlockSpec(memory_space=pltpu.MemorySpace.HBM),
            pl.BlockSpec(memory_space=pltpu.VMEM),
            pl.BlockSpec(memory_space=pltpu.VMEM),
        ],
        out_specs=pl.BlockSpec(memory_space=pltpu.VMEM),
        scratch_shapes=[
            pltpu.VMEM((2, SUB + HALO, cin), jnp.float32),
            pltpu.SemaphoreType.DMA((2,)),
            pltpu.VMEM((2, HALF, COUT), jnp.float32),
            pltpu.VMEM((2, HALF, COUT), jnp.float32),
            pltpu.SemaphoreType.DMA((2,)),
            pltpu.SemaphoreType.DMA((2, 2)),
            pltpu.SemaphoreType.DMA((2, STEPS)),
            pltpu.SemaphoreType.REGULAR((2,)),
        ],
        compiler_params=pltpu.CompilerParams(
            collective_id=0, vmem_limit_bytes=63 * 1024 * 1024),
    )(x, k, Wp)
    return out.reshape(B, SEQ, COUT)


# device time: 271850 ns/iter; 2.6980x vs baseline; 1.5597x over previous
import jax
import jax.numpy as jnp
from jax import lax
from jax.experimental import pallas as pl
from jax.experimental.pallas import tpu as pltpu

N_DEV = 8
B = 2
SEQ = 4096
TAPS = 4
COUT = 1024
N_CHUNK = 8
ROWS = B * SEQ
CH_ROWS = ROWS // N_CHUNK
HALF = CH_ROWS // 2
STEPS = N_DEV - 1

SUB = 512
NSUB = ROWS // SUB
HALO = 8


def kernel(x, k, Wp):
    cin = x.shape[2]

    def body(x_ref, k_ref, w_ref, out_ref,
             stage_ref, copy_sems, send_refs, rs_recv, ag_recv,
             send_sems, rs_sems, ag_sems, credit_sems):
        my = lax.axis_index("i")
        left = lax.rem(my + N_DEV - 1, N_DEV)
        right = lax.rem(my + 1, N_DEV)

        barrier = pltpu.get_barrier_semaphore()
        for nbr in (left, right):
            pl.semaphore_signal(barrier, inc=1, device_id=(nbr,),
                                device_id_type=pl.DeviceIdType.MESH)
        pl.semaphore_wait(barrier, 2)

        def issue_copy(n):
            slot = n % 2
            b, j = divmod(n, NSUB // B)
            s0 = j * SUB
            if j == 0:
                stage_ref[slot, 0:HALO, :] = jnp.zeros((HALO, cin), jnp.float32)
                cp = pltpu.make_async_copy(
                    x_ref.at[b, pl.ds(0, SUB), :],
                    stage_ref.at[slot, pl.ds(HALO, SUB), :],
                    copy_sems.at[slot],
                )
            else:
                cp = pltpu.make_async_copy(
                    x_ref.at[b, pl.ds(s0 - HALO, SUB + HALO), :],
                    stage_ref.at[slot, :, :],
                    copy_sems.at[slot],
                )
            cp.start()
            return cp

        kv = k_ref[:, :]
        w = w_ref[:, :]
        cp = issue_copy(0)
        for n in range(NSUB):
            cp.wait()
            if n + 1 < NSUB:
                cp = issue_copy(n + 1)
            slot = n % 2
            xp = stage_ref[slot]
            y = (xp[HALO:HALO + SUB] * kv[3]
                 + xp[HALO - 1:HALO - 1 + SUB] * kv[2]
                 + xp[HALO - 2:HALO - 2 + SUB] * kv[1]
                 + xp[HALO - 3:HALO - 3 + SUB] * kv[0])
            a = y * jax.nn.sigmoid(y)
            c, h = divmod(n, CH_ROWS // SUB)
            out_ref[c, pl.ds(h * SUB, SUB), :] = jnp.dot(
                a, w, preferred_element_type=jnp.float32)

        def hop(sc0, sc1, recv_bufs, recv_sems, slot):
            send_refs[0] = out_ref[sc0, pl.ds(0, HALF), :].astype(jnp.bfloat16)
            send_refs[1] = out_ref[sc1, pl.ds(HALF, HALF), :].astype(jnp.bfloat16)
            cw = pltpu.make_async_remote_copy(
                src_ref=send_refs.at[0],
                dst_ref=recv_bufs.at[0, slot],
                send_sem=send_sems.at[0],
                recv_sem=recv_sems.at[0, slot],
                device_id=(right,),
                device_id_type=pl.DeviceIdType.MESH,
            )
            ccw = pltpu.make_async_remote_copy(
                src_ref=send_refs.at[1],
                dst_ref=recv_bufs.at[1, slot],
                send_sem=send_sems.at[1],
                recv_sem=recv_sems.at[1, slot],
                device_id=(left,),
                device_id_type=pl.DeviceIdType.MESH,
            )
            cw.start()
            ccw.start()
            cw.wait()
            ccw.wait()

        def credit_wait(phase):
            pl.semaphore_wait(credit_sems.at[phase, 0], 1)
            pl.semaphore_wait(credit_sems.at[phase, 1], 1)

        def credit_signal(phase):
            pl.semaphore_signal(credit_sems.at[phase, 0], inc=1,
                                device_id=(left,),
                                device_id_type=pl.DeviceIdType.MESH)
            pl.semaphore_signal(credit_sems.at[phase, 1], inc=1,
                                device_id=(right,),
                                device_id_type=pl.DeviceIdType.MESH)

        for s in range(STEPS):
            slot = s % 2
            sc0 = lax.rem(my - s + N_DEV, N_DEV)
            rc0 = lax.rem(my - s - 1 + N_DEV, N_DEV)
            sc1 = lax.rem(my + s, N_DEV)
            rc1 = lax.rem(my + s + 1, N_DEV)
            if s >= 2:
                credit_wait(0)
            hop(sc0, sc1, rs_recv, rs_sems, slot)
            out_ref[rc0, pl.ds(0, HALF), :] = (
                out_ref[rc0, pl.ds(0, HALF), :]
                + rs_recv[0, slot].astype(jnp.float32))
            out_ref[rc1, pl.ds(HALF, HALF), :] = (
                out_ref[rc1, pl.ds(HALF, HALF), :]
                + rs_recv[1, slot].astype(jnp.float32))
            if s < STEPS - 2:
                credit_signal(0)

        for s in range(STEPS):
            slot = s % 2
            sc0 = lax.rem(my + 1 - s + N_DEV, N_DEV)
            rc0 = lax.rem(my - s + N_DEV, N_DEV)
            sc1 = lax.rem(my - 1 + s + N_DEV, N_DEV)
            rc1 = lax.rem(my + s, N_DEV)
            if s >= 2:
                credit_wait(1)
            hop(sc0, sc1, ag_recv, ag_sems, slot)
            out_ref[rc0, pl.ds(0, HALF), :] = (
                ag_recv[0, slot].astype(jnp.float32))
            out_ref[rc1, pl.ds(HALF, HALF), :] = (
                ag_recv[1, slot].astype(jnp.float32))
            if s < STEPS - 2:
                credit_signal(1)

    out = pl.pallas_call(
        body,
        out_shape=jax.ShapeDtypeStruct((N_CHUNK, CH_ROWS, COUT), jnp.float32),
        in_specs=[
            pl.BlockSpec(memory_space=pltpu.MemorySpace.HBM),
            pl.BlockSpec(memory_space=pltpu.VMEM),
            pl.BlockSpec(memory_space=pltpu.VMEM),
        ],
        out_specs=pl.BlockSpec(memory_space=pltpu.VMEM),
        scratch_shapes=[
            pltpu.VMEM((2, SUB + HALO, cin), jnp.float32),
            pltpu.SemaphoreType.DMA((2,)),
            pltpu.VMEM((2, HALF, COUT), jnp.bfloat16),
            pltpu.VMEM((2, 2, HALF, COUT), jnp.bfloat16),
            pltpu.VMEM((2, 2, HALF, COUT), jnp.bfloat16),
            pltpu.SemaphoreType.DMA((2,)),
            pltpu.SemaphoreType.DMA((2, 2)),
            pltpu.SemaphoreType.DMA((2, 2)),
            pltpu.SemaphoreType.REGULAR((2, 2)),
        ],
        compiler_params=pltpu.CompilerParams(
            collective_id=0, vmem_limit_bytes=63 * 1024 * 1024),
    )(x, k, Wp)
    return out.reshape(B, SEQ, COUT)


# device time: 245508 ns/iter; 2.9875x vs baseline; 1.1073x over previous
import jax
import jax.numpy as jnp
from jax import lax
from jax.experimental import pallas as pl
from jax.experimental.pallas import tpu as pltpu

N_DEV = 8
B = 2
SEQ = 4096
TAPS = 4
COUT = 1024
N_CHUNK = 8
ROWS = B * SEQ
CH_ROWS = ROWS // N_CHUNK
HALF = CH_ROWS // 2
STEPS = N_DEV - 1
SUB = HALF
HALO = 8
SEQ_PER_CHUNK = 4


def kernel(x, k, Wp):
    cin = x.shape[2]

    def body(x_ref, k_ref, w_ref, out_ref,
             stage_ref, copy_sems, send_refs, rs_recv, ag_recv,
             send_sems, rs_sems, ag_sems, credit_sems):
        my = lax.axis_index("i")
        left = lax.rem(my + N_DEV - 1, N_DEV)
        right = lax.rem(my + 1, N_DEV)

        barrier = pltpu.get_barrier_semaphore()
        for nbr in (left, right):
            pl.semaphore_signal(barrier, inc=1, device_id=(nbr,),
                                device_id_type=pl.DeviceIdType.MESH)
        pl.semaphore_wait(barrier, 2)

        kv = k_ref[:, :]
        w = w_ref[:, :]

        def start_load(g, h, slot):
            b = lax.div(g, SEQ_PER_CHUNK)
            s0 = pl.multiple_of(
                lax.rem(g, SEQ_PER_CHUNK) * CH_ROWS + h * SUB, SUB)
            main = pltpu.make_async_copy(
                x_ref.at[b, pl.ds(s0, SUB), :],
                stage_ref.at[slot, pl.ds(HALO, SUB), :],
                copy_sems.at[slot],
            )
            halo = pltpu.make_async_copy(
                x_ref.at[b, pl.ds(
                    pl.multiple_of(jnp.maximum(s0 - HALO, 0), HALO), HALO), :],
                stage_ref.at[slot, pl.ds(0, HALO), :],
                copy_sems.at[slot],
            )
            main.start()
            halo.start()
            return main, halo, s0

        def finish_load(ld, slot):
            main, halo, s0 = ld
            main.wait()
            halo.wait()

            @pl.when(s0 == 0)
            def _():
                stage_ref[slot, 0:HALO, :] = jnp.zeros((HALO, cin), jnp.float32)

        def compute_half(slot):
            xp = stage_ref[slot]
            y = (xp[HALO:HALO + SUB] * kv[3]
                 + xp[HALO - 1:HALO - 1 + SUB] * kv[2]
                 + xp[HALO - 2:HALO - 2 + SUB] * kv[1]
                 + xp[HALO - 3:HALO - 3 + SUB] * kv[0])
            a = y * jax.nn.sigmoid(y)
            return jnp.dot(a, w, preferred_element_type=jnp.float32)

        def credit_wait(phase):
            pl.semaphore_wait(credit_sems.at[phase, 0], 1)
            pl.semaphore_wait(credit_sems.at[phase, 1], 1)

        def credit_signal(phase):
            pl.semaphore_signal(credit_sems.at[phase, 0], inc=1,
                                device_id=(left,),
                                device_id_type=pl.DeviceIdType.MESH)
            pl.semaphore_signal(credit_sems.at[phase, 1], inc=1,
                                device_id=(right,),
                                device_id_type=pl.DeviceIdType.MESH)

        ld0 = start_load(my, 0, 0)
        ld1 = start_load(my, 1, 1)
        finish_load(ld0, 0)
        send_refs[0] = compute_half(0).astype(jnp.bfloat16)
        finish_load(ld1, 1)
        send_refs[1] = compute_half(1).astype(jnp.bfloat16)

        for s in range(STEPS):
            slot = s % 2
            rc0 = lax.rem(my - s - 1 + N_DEV, N_DEV)
            rc1 = lax.rem(my + s + 1, N_DEV)
            if s >= 2:
                credit_wait(0)
            cw = pltpu.make_async_remote_copy(
                src_ref=send_refs.at[0],
                dst_ref=rs_recv.at[0, slot],
                send_sem=send_sems.at[0],
                recv_sem=rs_sems.at[0, slot],
                device_id=(right,),
                device_id_type=pl.DeviceIdType.MESH,
            )
            ccw = pltpu.make_async_remote_copy(
                src_ref=send_refs.at[1],
                dst_ref=rs_recv.at[1, slot],
                send_sem=send_sems.at[1],
                recv_sem=rs_sems.at[1, slot],
                device_id=(left,),
                device_id_type=pl.DeviceIdType.MESH,
            )
            cw.start()
            ccw.start()
            ld0 = start_load(rc0, 0, 0)
            ld1 = start_load(rc1, 1, 1)
            finish_load(ld0, 0)
            out_ref[rc0, pl.ds(0, HALF), :] = compute_half(0)
            finish_load(ld1, 1)
            out_ref[rc1, pl.ds(HALF, HALF), :] = compute_half(1)
            cw.wait()
            ccw.wait()
            acc0 = (out_ref[rc0, pl.ds(0, HALF), :]
                    + rs_recv[0, slot].astype(jnp.float32))
            out_ref[rc0, pl.ds(0, HALF), :] = acc0
            send_refs[0] = acc0.astype(jnp.bfloat16)
            acc1 = (out_ref[rc1, pl.ds(HALF, HALF), :]
                    + rs_recv[1, slot].astype(jnp.float32))
            out_ref[rc1, pl.ds(HALF, HALF), :] = acc1
            send_refs[1] = acc1.astype(jnp.bfloat16)
            if s < STEPS - 2:
                credit_signal(0)

        for s in range(STEPS):
            slot = s % 2
            prev = (s - 1) % 2
            if s >= 2:
                credit_wait(1)
            src0 = send_refs.at[0] if s == 0 else ag_recv.at[0, prev]
            src1 = send_refs.at[1] if s == 0 else ag_recv.at[1, prev]
            cw = pltpu.make_async_remote_copy(
                src_ref=src0,
                dst_ref=ag_recv.at[0, slot],
                send_sem=send_sems.at[0],
                recv_sem=ag_sems.at[0, slot],
                device_id=(right,),
                device_id_type=pl.DeviceIdType.MESH,
            )
            ccw = pltpu.make_async_remote_copy(
                src_ref=src1,
                dst_ref=ag_recv.at[1, slot],
                send_sem=send_sems.at[1],
                recv_sem=ag_sems.at[1, slot],
                device_id=(left,),
                device_id_type=pl.DeviceIdType.MESH,
            )
            cw.start()
            ccw.start()
            if s >= 1:
                pc0 = lax.rem(my - s + 1 + N_DEV, N_DEV)
                pc1 = lax.rem(my + s - 1, N_DEV)
                out_ref[pc0, pl.ds(0, HALF), :] = (
                    ag_recv[0, prev].astype(jnp.float32))
                out_ref[pc1, pl.ds(HALF, HALF), :] = (
                    ag_recv[1, prev].astype(jnp.float32))
            cw.wait()
            ccw.wait()
            if 1 <= s <= STEPS - 2:
                credit_signal(1)
        lc0 = lax.rem(my - STEPS + 1 + N_DEV, N_DEV)
        lc1 = lax.rem(my + STEPS - 1, N_DEV)
        lslot = (STEPS - 1) % 2
        out_ref[lc0, pl.ds(0, HALF), :] = ag_recv[0, lslot].astype(jnp.float32)
        out_ref[lc1, pl.ds(HALF, HALF), :] = ag_recv[1, lslot].astype(jnp.float32)

    out = pl.pallas_call(
        body,
        out_shape=jax.ShapeDtypeStruct((N_CHUNK, CH_ROWS, COUT), jnp.float32),
        in_specs=[
            pl.BlockSpec(memory_space=pltpu.MemorySpace.HBM),
            pl.BlockSpec(memory_space=pltpu.VMEM),
            pl.BlockSpec(memory_space=pltpu.VMEM),
        ],
        out_specs=pl.BlockSpec(memory_space=pltpu.VMEM),
        scratch_shapes=[
            pltpu.VMEM((2, SUB + HALO, cin), jnp.float32),
            pltpu.SemaphoreType.DMA((2,)),
            pltpu.VMEM((2, HALF, COUT), jnp.bfloat16),
            pltpu.VMEM((2, 2, HALF, COUT), jnp.bfloat16),
            pltpu.VMEM((2, 2, HALF, COUT), jnp.bfloat16),
            pltpu.SemaphoreType.DMA((2,)),
            pltpu.SemaphoreType.DMA((2, 2)),
            pltpu.SemaphoreType.DMA((2, 2)),
            pltpu.SemaphoreType.REGULAR((2, 2)),
        ],
        compiler_params=pltpu.CompilerParams(
            collective_id=0, vmem_limit_bytes=63 * 1024 * 1024),
    )(x, k, Wp)
    return out.reshape(B, SEQ, COUT)


# device time: 236457 ns/iter; 3.1018x vs baseline; 1.0383x over previous
import jax
import jax.numpy as jnp
from jax import lax
from jax.experimental import pallas as pl
from jax.experimental.pallas import tpu as pltpu

N_DEV = 8
B = 2
SEQ = 4096
TAPS = 4
COUT = 1024
N_CHUNK = 8
ROWS = B * SEQ
CH_ROWS = ROWS // N_CHUNK
HALF = CH_ROWS // 2
STEPS = N_DEV - 1
SUB = HALF
HALO = 8
SEQ_PER_CHUNK = 4


def kernel(x, k, Wp):
    cin = x.shape[2]

    def body(x_ref, k_ref, w_ref, out_ref,
             stage_ref, copy_sems, send_refs, rs_recv, ag_recv,
             send_sems, rs_sems, ag_sems, credit_sems):
        def perm(p):
            return jnp.where(p < 4, p, 11 - p)

        pos = lax.axis_index("i")
        my = perm(pos)
        left = perm(lax.rem(my + N_DEV - 1, N_DEV))
        right = perm(lax.rem(my + 1, N_DEV))

        barrier = pltpu.get_barrier_semaphore()
        for nbr in (left, right):
            pl.semaphore_signal(barrier, inc=1, device_id=(nbr,),
                                device_id_type=pl.DeviceIdType.MESH)
        pl.semaphore_wait(barrier, 2)

        kv = k_ref[:, :]
        w = w_ref[:, :]

        def start_load(g, h, slot):
            b = lax.div(g, SEQ_PER_CHUNK)
            s0 = pl.multiple_of(
                lax.rem(g, SEQ_PER_CHUNK) * CH_ROWS + h * SUB, SUB)
            main = pltpu.make_async_copy(
                x_ref.at[b, pl.ds(s0, SUB), :],
                stage_ref.at[slot, pl.ds(HALO, SUB), :],
                copy_sems.at[slot],
            )
            halo = pltpu.make_async_copy(
                x_ref.at[b, pl.ds(
                    pl.multiple_of(jnp.maximum(s0 - HALO, 0), HALO), HALO), :],
                stage_ref.at[slot, pl.ds(0, HALO), :],
                copy_sems.at[slot],
            )
            main.start()
            halo.start()
            return main, halo, s0

        def finish_load(ld, slot):
            main, halo, s0 = ld
            main.wait()
            halo.wait()

            @pl.when(s0 == 0)
            def _():
                stage_ref[slot, 0:HALO, :] = jnp.zeros((HALO, cin), jnp.float32)

        def compute_half(slot):
            xp = stage_ref[slot]
            y = (xp[HALO:HALO + SUB] * kv[3]
                 + xp[HALO - 1:HALO - 1 + SUB] * kv[2]
                 + xp[HALO - 2:HALO - 2 + SUB] * kv[1]
                 + xp[HALO - 3:HALO - 3 + SUB] * kv[0])
            a = y * jax.nn.sigmoid(y)
            return jnp.dot(a, w, preferred_element_type=jnp.float32)

        def credit_wait(phase):
            pl.semaphore_wait(credit_sems.at[phase, 0], 1)
            pl.semaphore_wait(credit_sems.at[phase, 1], 1)

        def credit_signal(phase):
            pl.semaphore_signal(credit_sems.at[phase, 0], inc=1,
                                device_id=(left,),
                                device_id_type=pl.DeviceIdType.MESH)
            pl.semaphore_signal(credit_sems.at[phase, 1], inc=1,
                                device_id=(right,),
                                device_id_type=pl.DeviceIdType.MESH)

        ld0 = start_load(my, 0, 0)
        ld1 = start_load(my, 1, 1)
        finish_load(ld0, 0)
        send_refs[0] = compute_half(0).astype(jnp.bfloat16)
        finish_load(ld1, 1)
        send_refs[1] = compute_half(1).astype(jnp.bfloat16)

        for s in range(STEPS):
            slot = s % 2
            rc0 = lax.rem(my - s - 1 + N_DEV, N_DEV)
            rc1 = lax.rem(my + s + 1, N_DEV)
            if s >= 2:
                credit_wait(0)
            cw = pltpu.make_async_remote_copy(
                src_ref=send_refs.at[0],
                dst_ref=rs_recv.at[0, slot],
                send_sem=send_sems.at[0],
                recv_sem=rs_sems.at[0, slot],
                device_id=(right,),
                device_id_type=pl.DeviceIdType.MESH,
            )
            ccw = pltpu.make_async_remote_copy(
                src_ref=send_refs.at[1],
                dst_ref=rs_recv.at[1, slot],
                send_sem=send_sems.at[1],
                recv_sem=rs_sems.at[1, slot],
                device_id=(left,),
                device_id_type=pl.DeviceIdType.MESH,
            )
            cw.start()
            ccw.start()
            ld0 = start_load(rc0, 0, 0)
            ld1 = start_load(rc1, 1, 1)
            finish_load(ld0, 0)
            out_ref[rc0, pl.ds(0, HALF), :] = compute_half(0)
            finish_load(ld1, 1)
            out_ref[rc1, pl.ds(HALF, HALF), :] = compute_half(1)
            cw.wait()
            ccw.wait()
            acc0 = (out_ref[rc0, pl.ds(0, HALF), :]
                    + rs_recv[0, slot].astype(jnp.float32))
            out_ref[rc0, pl.ds(0, HALF), :] = acc0
            send_refs[0] = acc0.astype(jnp.bfloat16)
            acc1 = (out_ref[rc1, pl.ds(HALF, HALF), :]
                    + rs_recv[1, slot].astype(jnp.float32))
            out_ref[rc1, pl.ds(HALF, HALF), :] = acc1
            send_refs[1] = acc1.astype(jnp.bfloat16)
            if s < STEPS - 2:
                credit_signal(0)

        for s in range(STEPS):
            slot = s % 2
            prev = (s - 1) % 2
            if s >= 2:
                credit_wait(1)
            src0 = send_refs.at[0] if s == 0 else ag_recv.at[0, prev]
            src1 = send_refs.at[1] if s == 0 else ag_recv.at[1, prev]
            cw = pltpu.make_async_remote_copy(
                src_ref=src0,
                dst_ref=ag_recv.at[0, slot],
                send_sem=send_sems.at[0],
                recv_sem=ag_sems.at[0, slot],
                device_id=(right,),
                device_id_type=pl.DeviceIdType.MESH,
            )
            ccw = pltpu.make_async_remote_copy(
                src_ref=src1,
                dst_ref=ag_recv.at[1, slot],
                send_sem=send_sems.at[1],
                recv_sem=ag_sems.at[1, slot],
                device_id=(left,),
                device_id_type=pl.DeviceIdType.MESH,
            )
            cw.start()
            ccw.start()
            if s >= 1:
                pc0 = lax.rem(my - s + 1 + N_DEV, N_DEV)
                pc1 = lax.rem(my + s - 1, N_DEV)
                out_ref[pc0, pl.ds(0, HALF), :] = (
                    ag_recv[0, prev].astype(jnp.float32))
                out_ref[pc1, pl.ds(HALF, HALF), :] = (
                    ag_recv[1, prev].astype(jnp.float32))
            cw.wait()
            ccw.wait()
            if 1 <= s <= STEPS - 2:
                credit_signal(1)
        lc0 = lax.rem(my - STEPS + 1 + N_DEV, N_DEV)
        lc1 = lax.rem(my + STEPS - 1, N_DEV)
        lslot = (STEPS - 1) % 2
        out_ref[lc0, pl.ds(0, HALF), :] = ag_recv[0, lslot].astype(jnp.float32)
        out_ref[lc1, pl.ds(HALF, HALF), :] = ag_recv[1, lslot].astype(jnp.float32)

    out = pl.pallas_call(
        body,
        out_shape=jax.ShapeDtypeStruct((N_CHUNK, CH_ROWS, COUT), jnp.float32),
        in_specs=[
            pl.BlockSpec(memory_space=pltpu.MemorySpace.HBM),
            pl.BlockSpec(memory_space=pltpu.VMEM),
            pl.BlockSpec(memory_space=pltpu.VMEM),
        ],
        out_specs=pl.BlockSpec(memory_space=pltpu.VMEM),
        scratch_shapes=[
            pltpu.VMEM((2, SUB + HALO, cin), jnp.float32),
            pltpu.SemaphoreType.DMA((2,)),
            pltpu.VMEM((2, HALF, COUT), jnp.bfloat16),
            pltpu.VMEM((2, 2, HALF, COUT), jnp.bfloat16),
            pltpu.VMEM((2, 2, HALF, COUT), jnp.bfloat16),
            pltpu.SemaphoreType.DMA((2,)),
            pltpu.SemaphoreType.DMA((2, 2)),
            pltpu.SemaphoreType.DMA((2, 2)),
            pltpu.SemaphoreType.REGULAR((2, 2)),
        ],
        compiler_params=pltpu.CompilerParams(
            collective_id=0, vmem_limit_bytes=63 * 1024 * 1024),
    )(x, k, Wp)
    return out.reshape(B, SEQ, COUT)


# device time: 207097 ns/iter; 3.5416x vs baseline; 1.1418x over previous
import jax
import jax.numpy as jnp
from jax import lax
from jax.experimental import pallas as pl
from jax.experimental.pallas import tpu as pltpu

N_DEV = 8
B = 2
SEQ = 4096
TAPS = 4
COUT = 1024
N_CHUNK = 8
ROWS = B * SEQ
CH_ROWS = ROWS // N_CHUNK
HALF = CH_ROWS // 2
QROWS = HALF // 2
STEPS = N_DEV - 1
SUB = HALF
HALO = 8
SEQ_PER_CHUNK = 4
AG_SLOTS = 3


def kernel(x, k, Wp):
    cin = x.shape[2]

    def body(x_ref, k_ref, w_ref, out_ref,
             stage_ref, copy_sems, srefs, rs_recv, ag_recv,
             rs_send_sems, ag_send_sems, rs_sems, ag_sems, credit_sems):
        def perm(p):
            return jnp.where(p < 4, p, 11 - p)

        pos = lax.axis_index("i")
        my = perm(pos)
        left = perm(lax.rem(my + N_DEV - 1, N_DEV))
        right = perm(lax.rem(my + 1, N_DEV))

        barrier = pltpu.get_barrier_semaphore()
        for nbr in (left, right):
            pl.semaphore_signal(barrier, inc=1, device_id=(nbr,),
                                device_id_type=pl.DeviceIdType.MESH)
        pl.semaphore_wait(barrier, 2)

        kv = k_ref[:, :]
        w16 = w_ref[:, :].astype(jnp.bfloat16)

        def start_load(g, h, slot):
            b = lax.div(g, SEQ_PER_CHUNK)
            s0 = pl.multiple_of(
                lax.rem(g, SEQ_PER_CHUNK) * CH_ROWS + h * SUB, SUB)
            main = pltpu.make_async_copy(
                x_ref.at[b, pl.ds(s0, SUB), :],
                stage_ref.at[slot, pl.ds(HALO, SUB), :],
                copy_sems.at[slot],
            )
            halo = pltpu.make_async_copy(
                x_ref.at[b, pl.ds(
                    pl.multiple_of(jnp.maximum(s0 - HALO, 0), HALO), HALO), :],
                stage_ref.at[slot, pl.ds(0, HALO), :],
                copy_sems.at[slot],
            )
            main.start()
            halo.start()
            return main, halo, s0

        def finish_load(ld, slot):
            main, halo, s0 = ld
            main.wait()
            halo.wait()

            @pl.when(s0 == 0)
            def _():
                stage_ref[slot, 0:HALO, :] = jnp.zeros((HALO, cin), jnp.float32)

        def compute_half(slot):
            xp = stage_ref[slot]
            y = (xp[HALO:HALO + SUB] * kv[3]
                 + xp[HALO - 1:HALO - 1 + SUB] * kv[2]
                 + xp[HALO - 2:HALO - 2 + SUB] * kv[1]
                 + xp[HALO - 3:HALO - 3 + SUB] * kv[0])
            a = y * jax.nn.sigmoid(y)
            return jnp.dot(a.astype(jnp.bfloat16), w16,
                           preferred_element_type=jnp.float32)

        def rs_desc(d, sub, slot):
            return pltpu.make_async_remote_copy(
                src_ref=srefs.at[d, sub],
                dst_ref=rs_recv.at[d, sub, slot],
                send_sem=rs_send_sems.at[d, sub],
                recv_sem=rs_sems.at[d, sub, slot],
                device_id=(right if d == 0 else left,),
                device_id_type=pl.DeviceIdType.MESH,
            )

        def ag_desc(d, sub, slot, src_ref):
            return pltpu.make_async_remote_copy(
                src_ref=src_ref,
                dst_ref=ag_recv.at[d, sub, slot],
                send_sem=ag_send_sems.at[d, sub],
                recv_sem=ag_sems.at[d, sub, slot],
                device_id=(right if d == 0 else left,),
                device_id_type=pl.DeviceIdType.MESH,
            )

        def credit_wait(phase):
            pl.semaphore_wait(credit_sems.at[phase, 0], 1)
            pl.semaphore_wait(credit_sems.at[phase, 1], 1)

        def credit_signal(phase):
            pl.semaphore_signal(credit_sems.at[phase, 0], inc=1,
                                device_id=(left,),
                                device_id_type=pl.DeviceIdType.MESH)
            pl.semaphore_signal(credit_sems.at[phase, 1], inc=1,
                                device_id=(right,),
                                device_id_type=pl.DeviceIdType.MESH)

        def rows(d, sub):
            return pl.ds(d * HALF + sub * QROWS, QROWS)

        ld0 = start_load(my, 0, 0)
        ld1 = start_load(my, 1, 1)
        finish_load(ld0, 0)
        p = compute_half(0)
        srefs[0, 0] = p[0:QROWS].astype(jnp.bfloat16)
        srefs[0, 1] = p[QROWS:HALF].astype(jnp.bfloat16)
        rs_desc(0, 0, 0).start()
        rs_desc(0, 1, 0).start()
        finish_load(ld1, 1)
        p = compute_half(1)
        srefs[1, 0] = p[0:QROWS].astype(jnp.bfloat16)
        srefs[1, 1] = p[QROWS:HALF].astype(jnp.bfloat16)
        rs_desc(1, 0, 0).start()
        rs_desc(1, 1, 0).start()
        rc0 = lax.rem(my - 1 + N_DEV, N_DEV)
        rc1 = lax.rem(my + 1, N_DEV)
        ld0 = start_load(rc0, 0, 0)
        ld1 = start_load(rc1, 1, 1)
        finish_load(ld0, 0)
        out_ref[rc0, pl.ds(0, HALF), :] = compute_half(0)
        finish_load(ld1, 1)
        out_ref[rc1, pl.ds(HALF, HALF), :] = compute_half(1)

        for s in range(STEPS):
            slot = s % 2
            rc = (lax.rem(my - s - 1 + N_DEV, N_DEV),
                  lax.rem(my + s + 1, N_DEV))
            if 1 <= s <= STEPS - 2:
                credit_wait(0)
            for sub in (0, 1):
                for d in (0, 1):
                    desc = rs_desc(d, sub, slot)
                    desc.wait()
                    acc = (out_ref[rc[d], rows(d, sub), :]
                           + rs_recv[d, sub, slot].astype(jnp.float32))
                    out_ref[rc[d], rows(d, sub), :] = acc
                    srefs[d, sub] = acc.astype(jnp.bfloat16)
                    if s < STEPS - 1:
                        rs_desc(d, sub, (s + 1) % 2).start()
            if s <= STEPS - 3:
                credit_signal(0)
            if s < STEPS - 1:
                nc0 = lax.rem(my - s - 2 + N_DEV, N_DEV)
                nc1 = lax.rem(my + s + 2, N_DEV)
                ld0 = start_load(nc0, 0, 0)
                ld1 = start_load(nc1, 1, 1)
                finish_load(ld0, 0)
                out_ref[nc0, pl.ds(0, HALF), :] = compute_half(0)
                finish_load(ld1, 1)
                out_ref[nc1, pl.ds(HALF, HALF), :] = compute_half(1)

        for d in (0, 1):
            for sub in (0, 1):
                ag_desc(d, sub, 0, srefs.at[d, sub]).start()
        for s in range(STEPS):
            slot = s % AG_SLOTS
            rcg = (lax.rem(my - s + N_DEV, N_DEV), lax.rem(my + s, N_DEV))
            if 2 <= s <= STEPS - 2:
                credit_wait(1)
            for sub in (0, 1):
                for d in (0, 1):
                    src = (srefs.at[d, sub] if s == 0
                           else ag_recv.at[d, sub, (s - 1) % AG_SLOTS])
                    desc = ag_desc(d, sub, slot, src)
                    desc.wait()
                    if s < STEPS - 1:
                        ag_desc(d, sub, (s + 1) % AG_SLOTS,
                                ag_recv.at[d, sub, slot]).start()
                    out_ref[rcg[d], rows(d, sub), :] = (
                        ag_recv[d, sub, slot].astype(jnp.float32))
            if 1 <= s <= STEPS - 3:
                credit_signal(1)

    out = pl.pallas_call(
        body,
        out_shape=jax.ShapeDtypeStruct((N_CHUNK, CH_ROWS, COUT), jnp.float32),
        in_specs=[
            pl.BlockSpec(memory_space=pltpu.MemorySpace.HBM),
            pl.BlockSpec(memory_space=pltpu.VMEM),
            pl.BlockSpec(memory_space=pltpu.VMEM),
        ],
        out_specs=pl.BlockSpec(memory_space=pltpu.VMEM),
        scratch_shapes=[
            pltpu.VMEM((2, SUB + HALO, cin), jnp.float32),
            pltpu.SemaphoreType.DMA((2,)),
            pltpu.VMEM((2, 2, QROWS, COUT), jnp.bfloat16),
            pltpu.VMEM((2, 2, 2, QROWS, COUT), jnp.bfloat16),
            pltpu.VMEM((2, 2, AG_SLOTS, QROWS, COUT), jnp.bfloat16),
            pltpu.SemaphoreType.DMA((2, 2)),
            pltpu.SemaphoreType.DMA((2, 2)),
            pltpu.SemaphoreType.DMA((2, 2, 2)),
            pltpu.SemaphoreType.DMA((2, 2, AG_SLOTS)),
            pltpu.SemaphoreType.REGULAR((2, 2)),
        ],
        compiler_params=pltpu.CompilerParams(
            collective_id=0, vmem_limit_bytes=63 * 1024 * 1024),
    )(x, k, Wp)
    return out.reshape(B, SEQ, COUT)


# device time: 196584 ns/iter; 3.7310x vs baseline; 1.0535x over previous
import jax
import jax.numpy as jnp
from jax import lax
from jax.experimental import pallas as pl
from jax.experimental.pallas import tpu as pltpu

N_DEV = 8
B = 2
SEQ = 4096
TAPS = 4
COUT = 1024
N_CHUNK = 8
ROWS = B * SEQ
CH_ROWS = ROWS // N_CHUNK
HALF = CH_ROWS // 2
QROWS = HALF // 2
STEPS = N_DEV - 1
SUB = HALF
HALO = 8
SEQ_PER_CHUNK = 4
AG_SLOTS = 3


def kernel(x, k, Wp):
    cin = x.shape[2]

    def body(x_ref, k_ref, w_ref, out_ref,
             stage_ref, copy_sems, rs_recv, ag_recv,
             rs_send_sems, ag_send_sems, rs_sems, ag_sems, credit_sems):
        def perm(p):
            return jnp.where(p < 4, p, 11 - p)

        pos = lax.axis_index("i")
        my = perm(pos)
        left = perm(lax.rem(my + N_DEV - 1, N_DEV))
        right = perm(lax.rem(my + 1, N_DEV))

        barrier = pltpu.get_barrier_semaphore()
        for nbr in (left, right):
            pl.semaphore_signal(barrier, inc=1, device_id=(nbr,),
                                device_id_type=pl.DeviceIdType.MESH)
        pl.semaphore_wait(barrier, 2)

        kv = k_ref[:, :]
        w16 = w_ref[:, :].astype(jnp.bfloat16)

        def start_load(g, h, slot):
            b = lax.div(g, SEQ_PER_CHUNK)
            s0 = pl.multiple_of(
                lax.rem(g, SEQ_PER_CHUNK) * CH_ROWS + h * SUB, SUB)
            main = pltpu.make_async_copy(
                x_ref.at[b, pl.ds(s0, SUB), :],
                stage_ref.at[slot, pl.ds(HALO, SUB), :],
                copy_sems.at[slot],
            )
            halo = pltpu.make_async_copy(
                x_ref.at[b, pl.ds(
                    pl.multiple_of(jnp.maximum(s0 - HALO, 0), HALO), HALO), :],
                stage_ref.at[slot, pl.ds(0, HALO), :],
                copy_sems.at[slot],
            )
            main.start()
            halo.start()
            return main, halo, s0

        def finish_load(ld, slot):
            main, halo, s0 = ld
            main.wait()
            halo.wait()

            @pl.when(s0 == 0)
            def _():
                stage_ref[slot, 0:HALO, :] = jnp.zeros((HALO, cin), jnp.float32)

        def compute_half(slot):
            xp = stage_ref[slot]
            y = (xp[HALO:HALO + SUB] * kv[3]
                 + xp[HALO - 1:HALO - 1 + SUB] * kv[2]
                 + xp[HALO - 2:HALO - 2 + SUB] * kv[1]
                 + xp[HALO - 3:HALO - 3 + SUB] * kv[0])
            a = y * jax.nn.sigmoid(y)
            return jnp.dot(a.astype(jnp.bfloat16), w16,
                           preferred_element_type=jnp.float32
                           ).astype(jnp.bfloat16)

        def rows(d, sub):
            return pl.ds(d * HALF + sub * QROWS, QROWS)

        def rs_desc(d, sub, slot, src):
            return pltpu.make_async_remote_copy(
                src_ref=src,
                dst_ref=rs_recv.at[d, sub, slot],
                send_sem=rs_send_sems.at[d, sub],
                recv_sem=rs_sems.at[d, sub, slot],
                device_id=(right if d == 0 else left,),
                device_id_type=pl.DeviceIdType.MESH,
            )

        def ag_desc(d, sub, slot, src):
            return pltpu.make_async_remote_copy(
                src_ref=src,
                dst_ref=ag_recv.at[d, sub, slot],
                send_sem=ag_send_sems.at[d, sub],
                recv_sem=ag_sems.at[d, sub, slot],
                device_id=(right if d == 0 else left,),
                device_id_type=pl.DeviceIdType.MESH,
            )

        def credit_wait(phase):
            pl.semaphore_wait(credit_sems.at[phase, 0], 1)
            pl.semaphore_wait(credit_sems.at[phase, 1], 1)

        def credit_signal(phase):
            pl.semaphore_signal(credit_sems.at[phase, 0], inc=1,
                                device_id=(left,),
                                device_id_type=pl.DeviceIdType.MESH)
            pl.semaphore_signal(credit_sems.at[phase, 1], inc=1,
                                device_id=(right,),
                                device_id_type=pl.DeviceIdType.MESH)

        ld0 = start_load(my, 0, 0)
        ld1 = start_load(my, 1, 1)
        finish_load(ld0, 0)
        out_ref[my, pl.ds(0, HALF), :] = compute_half(0)
        rs_desc(0, 0, 0, out_ref.at[my, rows(0, 0), :]).start()
        rs_desc(0, 1, 0, out_ref.at[my, rows(0, 1), :]).start()
        finish_load(ld1, 1)
        out_ref[my, pl.ds(HALF, HALF), :] = compute_half(1)
        rs_desc(1, 0, 0, out_ref.at[my, rows(1, 0), :]).start()
        rs_desc(1, 1, 0, out_ref.at[my, rows(1, 1), :]).start()
        rc0 = lax.rem(my - 1 + N_DEV, N_DEV)
        rc1 = lax.rem(my + 1, N_DEV)
        ld0 = start_load(rc0, 0, 0)
        ld1 = start_load(rc1, 1, 1)
        finish_load(ld0, 0)
        out_ref[rc0, pl.ds(0, HALF), :] = compute_half(0)
        finish_load(ld1, 1)
        out_ref[rc1, pl.ds(HALF, HALF), :] = compute_half(1)

        for s in range(STEPS):
            slot = s % 2
            rc = (lax.rem(my - s - 1 + N_DEV, N_DEV),
                  lax.rem(my + s + 1, N_DEV))
            if 1 <= s <= STEPS - 2:
                credit_wait(0)
            for sub in (0, 1):
                for d in (0, 1):
                    desc = rs_desc(d, sub, slot,
                                   out_ref.at[rc[d], rows(d, sub), :])
                    desc.wait()
                    acc = (out_ref[rc[d], rows(d, sub), :].astype(jnp.float32)
                           + rs_recv[d, sub, slot].astype(jnp.float32))
                    out_ref[rc[d], rows(d, sub), :] = acc.astype(jnp.bfloat16)
                    if s < STEPS - 1:
                        rs_desc(d, sub, (s + 1) % 2,
                                out_ref.at[rc[d], rows(d, sub), :]).start()
            if s <= STEPS - 3:
                credit_signal(0)
            if s < STEPS - 1:
                nc0 = lax.rem(my - s - 2 + N_DEV, N_DEV)
                nc1 = lax.rem(my + s + 2, N_DEV)
                ld0 = start_load(nc0, 0, 0)
                ld1 = start_load(nc1, 1, 1)
                finish_load(ld0, 0)
                out_ref[nc0, pl.ds(0, HALF), :] = compute_half(0)
                finish_load(ld1, 1)
                out_ref[nc1, pl.ds(HALF, HALF), :] = compute_half(1)

        oc = (rc1, rc0)
        for d in (0, 1):
            for sub in (0, 1):
                ag_desc(d, sub, 0, out_ref.at[oc[d], rows(d, sub), :]).start()
        for s in range(STEPS):
            slot = s % AG_SLOTS
            rcg = (lax.rem(my - s + N_DEV, N_DEV), lax.rem(my + s, N_DEV))
            if 2 <= s <= STEPS - 2:
                credit_wait(1)
            for sub in (0, 1):
                for d in (0, 1):
                    src = (out_ref.at[oc[d], rows(d, sub), :] if s == 0
                           else ag_recv.at[d, sub, (s - 1) % AG_SLOTS])
                    desc = ag_desc(d, sub, slot, src)
                    desc.wait()
                    if s < STEPS - 1:
                        ag_desc(d, sub, (s + 1) % AG_SLOTS,
                                ag_recv.at[d, sub, slot]).start()
                    out_ref[rcg[d], rows(d, sub), :] = ag_recv[d, sub, slot]
            if 1 <= s <= STEPS - 3:
                credit_signal(1)

    out = pl.pallas_call(
        body,
        out_shape=jax.ShapeDtypeStruct((N_CHUNK, CH_ROWS, COUT), jnp.bfloat16),
        in_specs=[
            pl.BlockSpec(memory_space=pltpu.MemorySpace.HBM),
            pl.BlockSpec(memory_space=pltpu.VMEM),
            pl.BlockSpec(memory_space=pltpu.VMEM),
        ],
        out_specs=pl.BlockSpec(memory_space=pltpu.VMEM),
        scratch_shapes=[
            pltpu.VMEM((2, SUB + HALO, cin), jnp.float32),
            pltpu.SemaphoreType.DMA((2,)),
            pltpu.VMEM((2, 2, 2, QROWS, COUT), jnp.bfloat16),
            pltpu.VMEM((2, 2, AG_SLOTS, QROWS, COUT), jnp.bfloat16),
            pltpu.SemaphoreType.DMA((2, 2)),
            pltpu.SemaphoreType.DMA((2, 2)),
            pltpu.SemaphoreType.DMA((2, 2, 2)),
            pltpu.SemaphoreType.DMA((2, 2, AG_SLOTS)),
            pltpu.SemaphoreType.REGULAR((2, 2)),
        ],
        compiler_params=pltpu.CompilerParams(
            collective_id=0, vmem_limit_bytes=63 * 1024 * 1024),
    )(x, k, Wp)
    return out.reshape(B, SEQ, COUT).astype(jnp.float32)
